# baseline (speedup 1.0000x reference)
import jax
import jax.numpy as jnp
from jax.experimental import pallas as pl

_DIM = 256
_NQ = 4
_K = 1024
_B = 8192
_BB = 512
_ILANES = 128


def _rvq_block(x_ref, cb_ref, q_ref, idx_ref):
    r = x_ref[...]
    ks = jax.lax.broadcasted_iota(jnp.int32, (_BB, _K), 1)
    for i in range(_NQ):
        w = cb_ref[i]
        x2 = jnp.sum(r * r, axis=-1, keepdims=True)
        w2 = jnp.sum(w * w, axis=-1)[None, :]
        s = jax.lax.dot_general(
            r, w, (((1,), (1,)), ((), ())),
            preferred_element_type=jnp.float32,
        )
        dist = (x2 + w2) - 2.0 * s
        m = jnp.min(dist, axis=-1, keepdims=True)
        idx = jnp.min(jnp.where(dist == m, ks, _K), axis=-1)
        oh = (ks == idx[:, None]).astype(jnp.float32)
        q = jax.lax.dot_general(
            oh, w, (((1,), (0,)), ((), ())),
            preferred_element_type=jnp.float32,
            precision=jax.lax.Precision.HIGHEST,
        )
        q_ref[:, i, :] = q
        idx_ref[:, i] = idx
        r = r - q


def kernel(x, codebooks):
    grid = (_B // _BB,)
    quantized, idx_pad = pl.pallas_call(
        _rvq_block,
        grid=grid,
        in_specs=[
            pl.BlockSpec((_BB, _DIM), lambda b: (b, 0)),
            pl.BlockSpec((_NQ, _K, _DIM), lambda b: (0, 0, 0)),
        ],
        out_specs=[
            pl.BlockSpec((_BB, _NQ, _DIM), lambda b: (b, 0, 0)),
            pl.BlockSpec((_BB, _ILANES), lambda b: (b, 0)),
        ],
        out_shape=[
            jax.ShapeDtypeStruct((_B, _NQ, _DIM), jnp.float32),
            jax.ShapeDtypeStruct((_B, _ILANES), jnp.int32),
        ],
    )(x, codebooks)
    indices = idx_pad[:, :_NQ]
    loss = jnp.zeros((), dtype=jnp.float32)
    return quantized, indices, loss


# speedup vs baseline: 2.6555x; 2.6555x over previous
import jax
import jax.numpy as jnp
from jax.experimental import pallas as pl

_DIM = 256
_NQ = 4
_K = 1024
_B = 8192
_BB = 1024
_ILANES = 128


def _rvq_block(x_ref, cb_ref, q_ref, idx_ref):
    r = x_ref[...]
    ks = jax.lax.broadcasted_iota(jnp.int32, (_BB, _K), 1)
    for i in range(_NQ):
        w = cb_ref[i]
        x2 = jnp.sum(r * r, axis=-1, keepdims=True)
        w2 = jnp.sum(w * w, axis=-1)[None, :]
        s = jax.lax.dot_general(
            r, w, (((1,), (1,)), ((), ())),
            preferred_element_type=jnp.float32,
        )
        dist = (x2 + w2) - 2.0 * s
        m = jnp.min(dist, axis=-1, keepdims=True)
        idx = jnp.min(jnp.where(dist == m, ks, _K), axis=-1)
        oh = (ks == idx[:, None]).astype(jnp.float32)
        q = jax.lax.dot_general(
            oh, w, (((1,), (0,)), ((), ())),
            preferred_element_type=jnp.float32,
        )
        q_ref[:, i, :] = q
        idx_ref[:, i] = idx
        r = r - q


def kernel(x, codebooks):
    grid = (_B // _BB,)
    quantized, idx_pad = pl.pallas_call(
        _rvq_block,
        grid=grid,
        in_specs=[
            pl.BlockSpec((_BB, _DIM), lambda b: (b, 0)),
            pl.BlockSpec((_NQ, _K, _DIM), lambda b: (0, 0, 0)),
        ],
        out_specs=[
            pl.BlockSpec((_BB, _NQ, _DIM), lambda b: (b, 0, 0)),
            pl.BlockSpec((_BB, _ILANES), lambda b: (b, 0)),
        ],
        out_shape=[
            jax.ShapeDtypeStruct((_B, _NQ, _DIM), jnp.float32),
            jax.ShapeDtypeStruct((_B, _ILANES), jnp.int32),
        ],
    )(x, codebooks)
    indices = idx_pad[:, :_NQ]
    loss = jnp.zeros((), dtype=jnp.float32)
    return quantized, indices, loss
